# baseline (speedup 1.0000x reference)
import dataclasses
import functools

import jax
import jax.numpy as jnp
from jax import lax
from jax.experimental import pallas as pl
from jax.experimental.pallas import tpu as pltpu
from jax.experimental.pallas import tpu_sc as plsc

ROWS = 8192
DIM = 2048
NUM_EXPERTS = 64
K = 8
CH = 4
CROWS = ROWS // CH
BM = 1024

NC, NS, L = 2, 16, 16
NW = NC * NS
RPW = CROWS // NW


def _score_kernel(x_ref, wt_ref, bias_ref, s_ref):
    logits = jnp.dot(
        x_ref[...], wt_ref[...],
        preferred_element_type=jnp.float32,
    )
    m = jnp.max(logits, axis=1, keepdims=True)
    e = jnp.exp(logits - m)
    p = e / jnp.sum(e, axis=1, keepdims=True)
    s_ref[...] = p + bias_ref[...]


def _scores_tc(x, wt, bias2, chunk):
    base = chunk * (CROWS // BM)
    return pl.pallas_call(
        _score_kernel,
        grid=(CROWS // BM,),
        in_specs=[
            pl.BlockSpec((BM, DIM), lambda i: (base + i, 0)),
            pl.BlockSpec((DIM, NUM_EXPERTS), lambda i: (0, 0)),
            pl.BlockSpec((1, NUM_EXPERTS), lambda i: (0, 0)),
        ],
        out_specs=pl.BlockSpec((BM, NUM_EXPERTS), lambda i: (i, 0)),
        out_shape=jax.ShapeDtypeStruct((CROWS, NUM_EXPERTS), jnp.float32),
    )(x, wt, bias2)


def _merge16(ka, va, kb, vb):
    rk = lax.rev(kb, (0,))
    rv = lax.rev(vb, (0,))
    m = ka >= rk
    mk = jnp.where(m, ka, rk)
    mv = jnp.where(m, va, rv)
    return plsc.sort_key_val(mk, mv, descending=True)


_SC_PARAMS = pltpu.CompilerParams()
if "needs_layout_passes" in pltpu.CompilerParams.__dataclass_fields__:
    _SC_PARAMS = dataclasses.replace(_SC_PARAMS, needs_layout_passes=False)


@functools.partial(
    pl.kernel,
    compiler_params=_SC_PARAMS,
    out_type=[
        jax.ShapeDtypeStruct((CROWS, L), jnp.float32),
        jax.ShapeDtypeStruct((CROWS, L), jnp.int32),
    ],
    mesh=plsc.VectorSubcoreMesh(core_axis_name="c", subcore_axis_name="s"),
    scratch_types=[
        pltpu.VMEM((RPW, NUM_EXPERTS), jnp.float32),
        pltpu.VMEM((RPW, L), jnp.float32),
        pltpu.VMEM((RPW, L), jnp.int32),
        pltpu.SemaphoreType.DMA,
    ],
)
def _topk_sc(s_hbm, w_hbm, i_hbm, s_v, w_v, i_v, sem):
    wid = lax.axis_index("s") * NC + lax.axis_index("c")
    base = wid * RPW
    pltpu.async_copy(s_hbm.at[pl.ds(base, RPW)], s_v, sem).wait()
    iota = lax.iota(jnp.int32, L)

    @pl.loop(0, RPW)
    def _(r):
        ks, vs = [], []
        for j in range(NUM_EXPERTS // L):
            kj = s_v[r, pl.ds(j * L, L)]
            kj, vj = plsc.sort_key_val(kj, iota + (j * L), descending=True)
            ks.append(kj)
            vs.append(vj)
        k01, v01 = _merge16(ks[0], vs[0], ks[1], vs[1])
        k23, v23 = _merge16(ks[2], vs[2], ks[3], vs[3])
        kf, vf = _merge16(k01, v01, k23, v23)
        w_v[r, :] = kf
        i_v[r, :] = vf

    pltpu.async_copy(w_v, w_hbm.at[pl.ds(base, RPW)], sem).wait()
    pltpu.async_copy(i_v, i_hbm.at[pl.ds(base, RPW)], sem).wait()


@jax.jit
def kernel(x, weights, bias):
    wt = weights.T
    bias2 = bias.reshape(1, NUM_EXPERTS)
    ws, idxs = [], []
    for c in range(CH):
        scores = _scores_tc(x, wt, bias2, c)
        w16, i16 = _topk_sc(scores)
        ws.append(w16)
        idxs.append(i16)
    w = jnp.concatenate(ws, axis=0)[:, :K]
    idx = jnp.concatenate(idxs, axis=0)[:, :K]
    return w, idx


# speedup vs baseline: 1.1792x; 1.1792x over previous
import dataclasses
import functools

import jax
import jax.numpy as jnp
from jax import lax
from jax.experimental import pallas as pl
from jax.experimental.pallas import tpu as pltpu
from jax.experimental.pallas import tpu_sc as plsc

ROWS = 8192
DIM = 2048
NUM_EXPERTS = 64
K = 8
BM = 1024

NC, NS, L = 2, 16, 16
NW = NC * NS
RPW = ROWS // NW


def _score_kernel(x_ref, wt_ref, bias_ref, s_ref):
    logits = jnp.dot(
        x_ref[...], wt_ref[...],
        preferred_element_type=jnp.float32,
    )
    m = jnp.max(logits, axis=1, keepdims=True)
    e = jnp.exp(logits - m)
    p = e / jnp.sum(e, axis=1, keepdims=True)
    s_ref[...] = p + bias_ref[...]


def _scores_tc(x, wt, bias2):
    return pl.pallas_call(
        _score_kernel,
        grid=(ROWS // BM,),
        in_specs=[
            pl.BlockSpec((BM, DIM), lambda i: (i, 0)),
            pl.BlockSpec((DIM, NUM_EXPERTS), lambda i: (0, 0)),
            pl.BlockSpec((1, NUM_EXPERTS), lambda i: (0, 0)),
        ],
        out_specs=pl.BlockSpec((BM, NUM_EXPERTS), lambda i: (i, 0)),
        out_shape=jax.ShapeDtypeStruct((ROWS, NUM_EXPERTS), jnp.float32),
    )(x, wt, bias2)


def _merge16(ka, va, kb, vb):
    rk = lax.rev(kb, (0,))
    rv = lax.rev(vb, (0,))
    m = ka >= rk
    mk = jnp.where(m, ka, rk)
    mv = jnp.where(m, va, rv)
    return plsc.sort_key_val(mk, mv, descending=True)


_SC_PARAMS = pltpu.CompilerParams()
if "needs_layout_passes" in pltpu.CompilerParams.__dataclass_fields__:
    _SC_PARAMS = dataclasses.replace(_SC_PARAMS, needs_layout_passes=False)


@functools.partial(
    pl.kernel,
    compiler_params=_SC_PARAMS,
    out_type=jax.ShapeDtypeStruct((ROWS, L), jnp.float32),
    mesh=plsc.VectorSubcoreMesh(core_axis_name="c", subcore_axis_name="s"),
    scratch_types=[
        pltpu.VMEM((RPW, NUM_EXPERTS), jnp.float32),
        pltpu.VMEM((RPW, L), jnp.float32),
        pltpu.SemaphoreType.DMA,
    ],
)
def _topk_sc(s_hbm, p_hbm, s_v, p_v, sem):
    wid = lax.axis_index("s") * NC + lax.axis_index("c")
    base = wid * RPW
    pltpu.async_copy(s_hbm.at[pl.ds(base, RPW)], s_v, sem).wait()
    iota = lax.iota(jnp.int32, L)
    shuf = jnp.maximum(iota - K, 0)
    low = iota < K

    @pl.loop(0, RPW)
    def _(r):
        ks, vs = [], []
        for j in range(NUM_EXPERTS // L):
            kj = s_v[r, pl.ds(j * L, L)]
            kj, vj = plsc.sort_key_val(kj, iota + (j * L), descending=True)
            ks.append(kj)
            vs.append(vj)
        k01, v01 = _merge16(ks[0], vs[0], ks[1], vs[1])
        k23, v23 = _merge16(ks[2], vs[2], ks[3], vs[3])
        kf, vf = _merge16(k01, v01, k23, v23)
        vf_sh = lax.gather(
            vf, shuf[:, None],
            dimension_numbers=lax.GatherDimensionNumbers(
                offset_dims=(), collapsed_slice_dims=(0,),
                start_index_map=(0,)),
            slice_sizes=(1,),
            mode=lax.GatherScatterMode.PROMISE_IN_BOUNDS)
        packed = jnp.where(low, kf, lax.bitcast_convert_type(vf_sh, jnp.float32))
        p_v[r, :] = packed

    pltpu.async_copy(p_v, p_hbm.at[pl.ds(base, RPW)], sem).wait()


@jax.jit
def kernel(x, weights, bias):
    wt = weights.T
    bias2 = bias.reshape(1, NUM_EXPERTS)
    scores = _scores_tc(x, wt, bias2)
    packed = _topk_sc(scores)
    w = packed[:, :K]
    idx = lax.bitcast_convert_type(packed[:, K:], jnp.int32)
    return w, idx


# speedup vs baseline: 1.3080x; 1.1092x over previous
import dataclasses
import functools

import jax
import jax.numpy as jnp
from jax import lax
from jax.experimental import pallas as pl
from jax.experimental.pallas import tpu as pltpu
from jax.experimental.pallas import tpu_sc as plsc

ROWS = 8192
DIM = 2048
NUM_EXPERTS = 64
K = 8
BM = 1024

NC, NS, L = 2, 16, 16
NW = NC * NS
RPW = ROWS // NW


def _score_kernel(x_ref, w_ref, bias_ref, s_ref):
    logits = lax.dot_general(
        x_ref[...], w_ref[...],
        dimension_numbers=(((1,), (1,)), ((), ())),
        preferred_element_type=jnp.float32,
    )
    m = jnp.max(logits, axis=1, keepdims=True)
    e = jnp.exp(logits - m)
    p = e / jnp.sum(e, axis=1, keepdims=True)
    s_ref[...] = p + bias_ref[...]


def _scores_tc(x, weights, bias2):
    return pl.pallas_call(
        _score_kernel,
        grid=(ROWS // BM,),
        in_specs=[
            pl.BlockSpec((BM, DIM), lambda i: (i, 0)),
            pl.BlockSpec((NUM_EXPERTS, DIM), lambda i: (0, 0)),
            pl.BlockSpec((1, NUM_EXPERTS), lambda i: (0, 0)),
        ],
        out_specs=pl.BlockSpec((BM, NUM_EXPERTS), lambda i: (i, 0)),
        out_shape=jax.ShapeDtypeStruct((ROWS, NUM_EXPERTS), jnp.float32),
    )(x, weights, bias2)


def _merge16(ka, va, kb, vb):
    rk = lax.rev(kb, (0,))
    rv = lax.rev(vb, (0,))
    m = ka >= rk
    mk = jnp.where(m, ka, rk)
    mv = jnp.where(m, va, rv)
    return plsc.sort_key_val(mk, mv, descending=True)


_SC_PARAMS = pltpu.CompilerParams()
if "needs_layout_passes" in pltpu.CompilerParams.__dataclass_fields__:
    _SC_PARAMS = dataclasses.replace(_SC_PARAMS, needs_layout_passes=False)


@functools.partial(
    pl.kernel,
    compiler_params=_SC_PARAMS,
    out_type=[
        jax.ShapeDtypeStruct((ROWS, L), jnp.float32),
        jax.ShapeDtypeStruct((ROWS, L), jnp.int32),
    ],
    mesh=plsc.VectorSubcoreMesh(core_axis_name="c", subcore_axis_name="s"),
    scratch_types=[
        pltpu.VMEM((RPW, NUM_EXPERTS), jnp.float32),
        pltpu.VMEM((RPW, L), jnp.float32),
        pltpu.VMEM((RPW, L), jnp.int32),
        pltpu.SemaphoreType.DMA,
        pltpu.SemaphoreType.DMA,
    ],
)
def _topk_sc(s_hbm, w_hbm, i_hbm, s_v, w_v, i_v, sem, sem2):
    wid = lax.axis_index("s") * NC + lax.axis_index("c")
    base = wid * RPW
    pltpu.async_copy(s_hbm.at[pl.ds(base, RPW)], s_v, sem).wait()
    iota = lax.iota(jnp.int32, L)

    @pl.loop(0, RPW)
    def _(r):
        ks, vs = [], []
        for j in range(NUM_EXPERTS // L):
            kj = s_v[r, pl.ds(j * L, L)]
            kj, vj = plsc.sort_key_val(kj, iota + (j * L), descending=True)
            ks.append(kj)
            vs.append(vj)
        k01, v01 = _merge16(ks[0], vs[0], ks[1], vs[1])
        k23, v23 = _merge16(ks[2], vs[2], ks[3], vs[3])
        kf, vf = _merge16(k01, v01, k23, v23)
        w_v[r, :] = kf
        i_v[r, :] = vf

    cw = pltpu.async_copy(w_v, w_hbm.at[pl.ds(base, RPW)], sem)
    ci = pltpu.async_copy(i_v, i_hbm.at[pl.ds(base, RPW)], sem2)
    cw.wait()
    ci.wait()


@jax.jit
def kernel(x, weights, bias):
    bias2 = bias.reshape(1, NUM_EXPERTS)
    scores = _scores_tc(x, weights, bias2)
    w16, i16 = _topk_sc(scores)
    return w16[:, :K], i16[:, :K]


# speedup vs baseline: 1.3251x; 1.0131x over previous
import dataclasses
import functools

import jax
import jax.numpy as jnp
from jax import lax
from jax.experimental import pallas as pl
from jax.experimental.pallas import tpu as pltpu
from jax.experimental.pallas import tpu_sc as plsc

ROWS = 8192
DIM = 2048
NUM_EXPERTS = 64
K = 8
BM = 1024

NC, NS, L = 2, 16, 16
NW = NC * NS
RPW = ROWS // NW


def _score_kernel(x_ref, w_ref, bias_ref, s_ref):
    logits = lax.dot_general(
        x_ref[...], w_ref[...],
        dimension_numbers=(((1,), (1,)), ((), ())),
        preferred_element_type=jnp.float32,
    )
    m = jnp.max(logits, axis=1, keepdims=True)
    e = jnp.exp(logits - m)
    p = e / jnp.sum(e, axis=1, keepdims=True)
    s_ref[...] = p + bias_ref[...]


def _scores_tc(x, weights, bias2):
    return pl.pallas_call(
        _score_kernel,
        grid=(ROWS // BM,),
        in_specs=[
            pl.BlockSpec((BM, DIM), lambda i: (i, 0)),
            pl.BlockSpec((NUM_EXPERTS, DIM), lambda i: (0, 0)),
            pl.BlockSpec((1, NUM_EXPERTS), lambda i: (0, 0)),
        ],
        out_specs=pl.BlockSpec((BM, NUM_EXPERTS), lambda i: (i, 0)),
        out_shape=jax.ShapeDtypeStruct((ROWS, NUM_EXPERTS), jnp.float32),
    )(x, weights, bias2)


def _merge16(ka, va, kb, vb):
    rk = lax.rev(kb, (0,))
    rv = lax.rev(vb, (0,))
    m = ka >= rk
    mk = jnp.where(m, ka, rk)
    mv = jnp.where(m, va, rv)
    return plsc.sort_key_val(mk, mv, descending=True)


_SC_PARAMS = pltpu.CompilerParams()
if "needs_layout_passes" in pltpu.CompilerParams.__dataclass_fields__:
    _SC_PARAMS = dataclasses.replace(_SC_PARAMS, needs_layout_passes=False)


@functools.partial(
    pl.kernel,
    compiler_params=_SC_PARAMS,
    out_type=[
        jax.ShapeDtypeStruct((ROWS, L), jnp.float32),
        jax.ShapeDtypeStruct((ROWS, L), jnp.int32),
    ],
    mesh=plsc.VectorSubcoreMesh(core_axis_name="c", subcore_axis_name="s"),
    scratch_types=[
        pltpu.VMEM((RPW, NUM_EXPERTS), jnp.float32),
        pltpu.VMEM((RPW, L), jnp.float32),
        pltpu.VMEM((RPW, L), jnp.int32),
        pltpu.SemaphoreType.DMA,
        pltpu.SemaphoreType.DMA,
    ],
)
def _topk_sc(s_hbm, w_hbm, i_hbm, s_v, w_v, i_v, sem, sem2):
    wid = lax.axis_index("s") * NC + lax.axis_index("c")
    base = wid * RPW
    half = RPW // 2
    c0 = pltpu.async_copy(
        s_hbm.at[pl.ds(base, half)], s_v.at[pl.ds(0, half)], sem)
    c1 = pltpu.async_copy(
        s_hbm.at[pl.ds(base + half, half)], s_v.at[pl.ds(half, half)], sem2)
    iota = lax.iota(jnp.int32, L)
    c0.wait()

    @pl.loop(0, half)
    def _(r):
        ks, vs = [], []
        for j in range(NUM_EXPERTS // L):
            kj = s_v[r, pl.ds(j * L, L)]
            kj, vj = plsc.sort_key_val(kj, iota + (j * L), descending=True)
            ks.append(kj)
            vs.append(vj)
        k01, v01 = _merge16(ks[0], vs[0], ks[1], vs[1])
        k23, v23 = _merge16(ks[2], vs[2], ks[3], vs[3])
        kf, vf = _merge16(k01, v01, k23, v23)
        w_v[r, :] = kf
        i_v[r, :] = vf

    c1.wait()

    @pl.loop(half, RPW)
    def _(r):
        ks, vs = [], []
        for j in range(NUM_EXPERTS // L):
            kj = s_v[r, pl.ds(j * L, L)]
            kj, vj = plsc.sort_key_val(kj, iota + (j * L), descending=True)
            ks.append(kj)
            vs.append(vj)
        k01, v01 = _merge16(ks[0], vs[0], ks[1], vs[1])
        k23, v23 = _merge16(ks[2], vs[2], ks[3], vs[3])
        kf, vf = _merge16(k01, v01, k23, v23)
        w_v[r, :] = kf
        i_v[r, :] = vf

    cw = pltpu.async_copy(w_v, w_hbm.at[pl.ds(base, RPW)], sem)
    ci = pltpu.async_copy(i_v, i_hbm.at[pl.ds(base, RPW)], sem2)
    cw.wait()
    ci.wait()


@jax.jit
def kernel(x, weights, bias):
    bias2 = bias.reshape(1, NUM_EXPERTS)
    scores = _scores_tc(x, weights, bias2)
    w16, i16 = _topk_sc(scores)
    return w16[:, :K], i16[:, :K]
